# baseline (speedup 1.0000x reference)
import jax
import jax.numpy as jnp
from jax import lax
from jax.experimental import pallas as pl
from jax.experimental.pallas import tpu as pltpu
from jax.experimental.pallas import tpu_sc as plsc

NC, NS, L = 2, 16, 16
LAT, LON = 361, 720
NVAR_IN, NVAR_OUT = 73, 10
PER_SC = NVAR_OUT // NC


def _gather_body(x_hbm, idx_hbm, out_hbm, vidx, shared, sem):
    c = lax.axis_index("c")
    s = lax.axis_index("s")
    pltpu.sync_copy(idx_hbm, vidx.at[pl.ds(0, NVAR_OUT)])

    @pl.when(s < PER_SC)
    def _():
        v = s * NC + c
        lane = lax.iota(jnp.int32, L)
        sv = jnp.sum(jnp.where(lane == v, vidx[...], 0))
        pltpu.async_copy(x_hbm.at[0, sv], shared.at[s], sem).wait()
        pltpu.async_copy(shared.at[s], out_hbm.at[0, v], sem).wait()


@jax.jit
def kernel(x, indexes):
    mesh = plsc.VectorSubcoreMesh(
        core_axis_name="c", subcore_axis_name="s", num_cores=NC, num_subcores=NS
    )
    return pl.kernel(
        _gather_body,
        out_type=jax.ShapeDtypeStruct((1, NVAR_OUT, LAT, LON), jnp.float32),
        mesh=mesh,
        scratch_types=[
            pltpu.VMEM((L,), jnp.int32),
            pltpu.VMEM_SHARED((PER_SC, LAT, LON), jnp.float32),
            pltpu.SemaphoreType.DMA,
        ],
        compiler_params=pltpu.CompilerParams(needs_layout_passes=False),
    )(x, indexes)
